# baseline (speedup 1.0000x reference)
import functools
import math

import jax
import jax.numpy as jnp
from jax import lax
from jax.experimental import pallas as pl
from jax.experimental.pallas import tpu as pltpu
from jax.experimental.pallas import tpu_sc as plsc

_B = 16384
_D = 128
_H = 256
_NC = 2
_NS = 16
_NW = _NC * _NS
_CS = (6144, 10240)
_BLK = 2048
_BN_INV = 1.0 / math.sqrt(1.0 + 1e-5)

_sc_mesh = plsc.VectorSubcoreMesh(core_axis_name="c", subcore_axis_name="s")


def _make_sc_gather(start, rows):
    bpw = rows // _NW

    @functools.partial(
        pl.kernel,
        mesh=_sc_mesh,
        out_type=jax.ShapeDtypeStruct((rows, _D), jnp.float32),
        scratch_types=[
            pltpu.VMEM((bpw,), jnp.int32),
            pltpu.VMEM((bpw, _D), jnp.float32),
            pltpu.SemaphoreType.DMA,
        ],
    )
    def _sc_gather(table_hbm, idx_hbm, out_hbm, idx_v, rows_v, sem):
        wid = lax.axis_index("s") * _NC + lax.axis_index("c")
        base = wid * bpw
        pltpu.sync_copy(idx_hbm.at[pl.ds(start + base, bpw)], idx_v)
        pltpu.async_copy(table_hbm.at[idx_v], rows_v, sem).wait()
        pltpu.sync_copy(rows_v, out_hbm.at[pl.ds(base, bpw)])

    return _sc_gather


_starts = [sum(_CS[:k]) for k in range(len(_CS))]
_sc_gathers = [_make_sc_gather(_starts[k], _CS[k]) for k in range(len(_CS))]


def _mlp_body(x_ref, w1_ref, b1_ref, g_ref, bt_ref, w2_ref, b2_ref, o_ref):
    h = jnp.dot(x_ref[...], w1_ref[...], preferred_element_type=jnp.float32)
    h = jnp.maximum(h + b1_ref[...], 0.0)
    h = h * (g_ref[...] * _BN_INV) + bt_ref[...]
    o_ref[...] = (
        jnp.dot(h, w2_ref[...], preferred_element_type=jnp.float32) + b2_ref[...]
    )


def _mlp_body_alias(x_ref, w1_ref, b1_ref, g_ref, bt_ref, w2_ref, b2_ref,
                    buf_ref, o_ref):
    _mlp_body(x_ref, w1_ref, b1_ref, g_ref, bt_ref, w2_ref, b2_ref, o_ref)


def _mlp_chunk(x, W1, b1, gamma, beta, W2, b2, buf, chunk):
    rows = _CS[chunk]
    base = _starts[chunk] // _BLK
    grid = (rows // _BLK,)
    common_specs = [
        pl.BlockSpec((_BLK, _D), lambda i: (i, 0)),
        pl.BlockSpec((_D, _H), lambda i: (0, 0)),
        pl.BlockSpec((1, _H), lambda i: (0, 0)),
        pl.BlockSpec((1, _H), lambda i: (0, 0)),
        pl.BlockSpec((1, _H), lambda i: (0, 0)),
        pl.BlockSpec((_H, _H), lambda i: (0, 0)),
        pl.BlockSpec((1, _H), lambda i: (0, 0)),
    ]
    out_spec = pl.BlockSpec((_BLK, _H), lambda i: (base + i, 0))
    out_shape = jax.ShapeDtypeStruct((_B, _H), jnp.float32)
    if buf is None:
        return pl.pallas_call(
            _mlp_body,
            grid=grid,
            in_specs=common_specs,
            out_specs=out_spec,
            out_shape=out_shape,
        )(x, W1, b1, gamma, beta, W2, b2)
    return pl.pallas_call(
        _mlp_body_alias,
        grid=grid,
        in_specs=common_specs + [pl.BlockSpec(memory_space=pl.ANY)],
        out_specs=out_spec,
        out_shape=out_shape,
        input_output_aliases={7: 0},
    )(x, W1, b1, gamma, beta, W2, b2, buf)


def kernel(target_ids, table, W1, b1, gamma, beta, W2, b2):
    idx = target_ids.astype(jnp.int32)
    b1r = b1.reshape(1, _H)
    gr = gamma.reshape(1, _H)
    btr = beta.reshape(1, _H)
    b2r = b2.reshape(1, _H)
    embs = [_sc_gathers[k](table, idx) for k in range(len(_CS))]
    buf = None
    for k in range(len(_CS)):
        buf = _mlp_chunk(embs[k], W1, b1r, gr, btr, W2, b2r, buf, k)
    return buf


# speedup vs baseline: 1.0578x; 1.0578x over previous
import functools
import math

import jax
import jax.numpy as jnp
from jax import lax
from jax.experimental import pallas as pl
from jax.experimental.pallas import tpu as pltpu
from jax.experimental.pallas import tpu_sc as plsc

_B = 16384
_D = 128
_H = 256
_NC = 2
_NS = 16
_NW = _NC * _NS
_CS = (8192, 8192)
_BLK = 2048
_BN_INV = 1.0 / math.sqrt(1.0 + 1e-5)

_sc_mesh = plsc.VectorSubcoreMesh(core_axis_name="c", subcore_axis_name="s")


def _make_sc_gather(start, rows):
    bpw = rows // _NW

    @functools.partial(
        pl.kernel,
        mesh=_sc_mesh,
        out_type=jax.ShapeDtypeStruct((rows, _D), jnp.float32),
        scratch_types=[
            pltpu.VMEM((bpw,), jnp.int32),
            pltpu.VMEM((bpw, _D), jnp.float32),
            pltpu.SemaphoreType.DMA,
        ],
    )
    def _sc_gather(table_hbm, idx_hbm, out_hbm, idx_v, rows_v, sem):
        wid = lax.axis_index("s") * _NC + lax.axis_index("c")
        base = wid * bpw
        pltpu.sync_copy(idx_hbm.at[pl.ds(start + base, bpw)], idx_v)
        pltpu.async_copy(table_hbm.at[idx_v], rows_v, sem).wait()
        pltpu.sync_copy(rows_v, out_hbm.at[pl.ds(base, bpw)])

    return _sc_gather


_starts = [sum(_CS[:k]) for k in range(len(_CS))]
_sc_gathers = [_make_sc_gather(_starts[k], _CS[k]) for k in range(len(_CS))]


def _mlp_body(x_ref, w1_ref, b1_ref, g_ref, bt_ref, w2_ref, b2_ref, o_ref):
    x = x_ref[...].astype(jnp.bfloat16)
    h = jnp.dot(x, w1_ref[...], preferred_element_type=jnp.float32)
    h = jnp.maximum(h + b1_ref[...], 0.0)
    h = (h * (g_ref[...] * _BN_INV) + bt_ref[...]).astype(jnp.bfloat16)
    o_ref[...] = (
        jnp.dot(h, w2_ref[...], preferred_element_type=jnp.float32) + b2_ref[...]
    )


def _mlp_body_alias(x_ref, w1_ref, b1_ref, g_ref, bt_ref, w2_ref, b2_ref,
                    buf_ref, o_ref):
    _mlp_body(x_ref, w1_ref, b1_ref, g_ref, bt_ref, w2_ref, b2_ref, o_ref)


def _mlp_chunk(x, W1, b1, gamma, beta, W2, b2, buf, chunk):
    rows = _CS[chunk]
    base = _starts[chunk] // _BLK
    grid = (rows // _BLK,)
    common_specs = [
        pl.BlockSpec((_BLK, _D), lambda i: (i, 0)),
        pl.BlockSpec((_D, _H), lambda i: (0, 0)),
        pl.BlockSpec((1, _H), lambda i: (0, 0)),
        pl.BlockSpec((1, _H), lambda i: (0, 0)),
        pl.BlockSpec((1, _H), lambda i: (0, 0)),
        pl.BlockSpec((_H, _H), lambda i: (0, 0)),
        pl.BlockSpec((1, _H), lambda i: (0, 0)),
    ]
    out_spec = pl.BlockSpec((_BLK, _H), lambda i: (base + i, 0))
    out_shape = jax.ShapeDtypeStruct((_B, _H), jnp.float32)
    if buf is None:
        return pl.pallas_call(
            _mlp_body,
            grid=grid,
            in_specs=common_specs,
            out_specs=out_spec,
            out_shape=out_shape,
        )(x, W1, b1, gamma, beta, W2, b2)
    return pl.pallas_call(
        _mlp_body_alias,
        grid=grid,
        in_specs=common_specs + [pl.BlockSpec(memory_space=pl.ANY)],
        out_specs=out_spec,
        out_shape=out_shape,
        input_output_aliases={7: 0},
    )(x, W1, b1, gamma, beta, W2, b2, buf)


def kernel(target_ids, table, W1, b1, gamma, beta, W2, b2):
    idx = target_ids.astype(jnp.int32)
    W1 = W1.astype(jnp.bfloat16)
    W2 = W2.astype(jnp.bfloat16)
    b1r = b1.reshape(1, _H)
    gr = gamma.reshape(1, _H)
    btr = beta.reshape(1, _H)
    b2r = b2.reshape(1, _H)
    embs = [_sc_gathers[k](table, idx) for k in range(len(_CS))]
    buf = None
    for k in range(len(_CS)):
        buf = _mlp_chunk(embs[k], W1, b1r, gr, btr, W2, b2r, buf, k)
    return buf
